# baseline (speedup 1.0000x reference)
import dataclasses
import functools

import jax
import jax.numpy as jnp
from jax import lax
from jax.experimental import pallas as pl
from jax.experimental.pallas import tpu as pltpu
from jax.experimental.pallas import tpu_sc as plsc

_NB = 4
_C = 512
_F = 128
_N = 256
_V = 100000
_CT = _NB * _C

_TOK_PER_TILE = _N // 16
_CHUNK = 2048
_NFULL = _V // _CHUNK
_PER_CORE = _NFULL // 2
_TAIL = _V - _NFULL * _CHUNK
_TAILP = 1792
_TAIL0 = 896
_TAIL1 = _TAIL - _TAIL0
_VPAD = _NFULL * _CHUNK + _TAILP
_MINI = 16


def _logits_body(x_ref, w_ref, b_ref, c_ref, out_ref, idx_ref):
    for b in range(_NB):
        xb = x_ref[:, b * _F:(b + 1) * _F]
        wb = w_ref[b]
        acc = lax.dot_general(
            xb, wb, (((1,), (1,)), ((), ())),
            preferred_element_type=jnp.float32)
        out_ref[:, b * _C:(b + 1) * _C] = acc + b_ref[b][None, :]
        idx_ref[b, :] = c_ref[b, :] + (b * _C)


def _compute_logits(x, w, bias, coords_pad):
    return pl.pallas_call(
        _logits_body,
        out_shape=(jax.ShapeDtypeStruct((_N, _CT), jnp.float32),
                   jax.ShapeDtypeStruct((_NB, _VPAD), jnp.int32)),
    )(x, w, bias, coords_pad)


def _decode_body(tt_hbm, idx_hbm, out_hbm,
                 slice_v, idx_v, out_v0, out_v1, tail_b, so0, so1):
    cid = lax.axis_index("c")
    sid = lax.axis_index("s")
    n0 = sid * _TOK_PER_TILE

    pltpu.sync_copy(tt_hbm.at[pl.ds(n0, _TOK_PER_TILE), :], slice_v)

    def _do_minis(out_v, width):
        @pl.loop(0, width // _MINI)
        def _mini(j):
            idxs = [idx_v[b, pl.ds(j * _MINI, _MINI)] for b in range(_NB)]
            for n in range(_TOK_PER_TILE):
                rows = jnp.full((_MINI,), n, jnp.int32)
                acc = plsc.load_gather(slice_v, [rows, idxs[0]])
                for b in range(1, _NB):
                    acc = acc + plsc.load_gather(slice_v, [rows, idxs[b]])
                out_v[n, pl.ds(j * _MINI, _MINI)] = acc

    bufs = (out_v0, out_v1)
    sems = (so0, so1)

    @pl.loop(0, _PER_CORE, step=2)
    def _chunk(i):
        for b in range(2):
            ii = i + b
            k = cid + 2 * ii
            start = pl.multiple_of(k * _CHUNK, _CHUNK)
            dst = out_hbm.at[pl.ds(n0, _TOK_PER_TILE), pl.ds(start, _CHUNK)]
            pltpu.sync_copy(idx_hbm.at[:, pl.ds(start, _CHUNK)], idx_v)

            @pl.when(ii >= 2)
            def _drain():
                pltpu.make_async_copy(bufs[b], dst, sems[b]).wait()

            _do_minis(bufs[b], _CHUNK)
            pltpu.async_copy(bufs[b], dst, sems[b])

    for b in range(2):
        dst = out_hbm.at[pl.ds(n0, _TOK_PER_TILE), pl.ds(0, _CHUNK)]
        pltpu.make_async_copy(bufs[b], dst, sems[b]).wait()

    tbase = _NFULL * _CHUNK

    @pl.when(cid == 0)
    def _tail0():
        pltpu.sync_copy(idx_hbm.at[:, pl.ds(tbase, _TAILP)],
                        idx_v.at[:, pl.ds(0, _TAILP)])
        _do_minis(out_v0, _TAIL0)
        pltpu.sync_copy(out_v0.at[:, pl.ds(0, _TAIL0)],
                        out_hbm.at[pl.ds(n0, _TOK_PER_TILE),
                                   pl.ds(tbase, _TAIL0)])

    @pl.when(cid == 1)
    def _tail1():
        pltpu.sync_copy(idx_hbm.at[:, pl.ds(tbase + _TAIL0, _TAILP - _TAIL0)],
                        idx_v.at[:, pl.ds(0, _TAILP - _TAIL0)])
        _do_minis(tail_b, _TAIL1)
        pltpu.sync_copy(tail_b,
                        out_hbm.at[pl.ds(n0, _TOK_PER_TILE),
                                   pl.ds(tbase + _TAIL0, _TAIL1)])


_SC_PARAMS = pltpu.CompilerParams()
if "needs_layout_passes" in pltpu.CompilerParams.__dataclass_fields__:
    _SC_PARAMS = dataclasses.replace(_SC_PARAMS, needs_layout_passes=False)
if "use_tc_tiling_on_sc" in pltpu.CompilerParams.__dataclass_fields__:
    _SC_PARAMS = dataclasses.replace(_SC_PARAMS, use_tc_tiling_on_sc=True)


@functools.partial(
    pl.kernel,
    out_type=jax.ShapeDtypeStruct((_N, _V), jnp.float32),
    compiler_params=_SC_PARAMS,
    mesh=plsc.VectorSubcoreMesh(core_axis_name="c", subcore_axis_name="s"),
    scratch_types=[
        pltpu.VMEM((_TOK_PER_TILE, _CT), jnp.float32),
        pltpu.VMEM((_NB, _CHUNK), jnp.int32),
        pltpu.VMEM((_TOK_PER_TILE, _CHUNK), jnp.float32),
        pltpu.VMEM((_TOK_PER_TILE, _CHUNK), jnp.float32),
        pltpu.VMEM((_TOK_PER_TILE, _TAIL1), jnp.float32),
        pltpu.SemaphoreType.DMA,
        pltpu.SemaphoreType.DMA,
    ],
)
def _decode(tt_hbm, idx_hbm, out_hbm,
            slice_v, idx_v, out_v0, out_v1, tail_b, so0, so1):
    _decode_body(tt_hbm, idx_hbm, out_hbm,
                 slice_v, idx_v, out_v0, out_v1, tail_b, so0, so1)


@jax.jit
def kernel(input, weight, bias, coordinates):
    coords_pad = jnp.concatenate(
        [coordinates,
         jnp.zeros((_NB, _VPAD - _V), jnp.int32)], axis=1)
    tt, idxp = _compute_logits(input, weight, bias, coords_pad)
    return _decode(tt, idxp)


# speedup vs baseline: 1.4176x; 1.4176x over previous
import dataclasses
import functools

import jax
import jax.numpy as jnp
from jax import lax
from jax.experimental import pallas as pl
from jax.experimental.pallas import tpu as pltpu
from jax.experimental.pallas import tpu_sc as plsc

_NB = 4
_C = 512
_F = 128
_N = 256
_NP = _N // 2
_V = 100000
_CT = _NB * _C

_PAIR_PER_TILE = _NP // 16
_CHUNK = 2048
_NFULL = _V // _CHUNK
_TAILH = 896
_VPAD = _NFULL * _CHUNK + 2 * _TAILH
_MINI = 16


def _logits_body(x_ref, w_ref, b_ref, c_ref, out_ref, idx_ref):
    for b in range(_NB):
        xb = x_ref[:, b * _F:(b + 1) * _F]
        wb = w_ref[b]
        acc = lax.dot_general(
            xb, wb, (((1,), (1,)), ((), ())),
            preferred_element_type=jnp.float32)
        acc = acc + b_ref[b][None, :]
        lo = lax.bitcast_convert_type(
            acc[:_NP].astype(jnp.bfloat16), jnp.uint16).astype(jnp.uint32)
        hi = lax.bitcast_convert_type(
            acc[_NP:].astype(jnp.bfloat16), jnp.uint16).astype(jnp.uint32)
        packed = lo | (hi << 16)
        out_ref[:, b * _C:(b + 1) * _C] = lax.bitcast_convert_type(
            packed, jnp.int32)
        idx_ref[b, :] = c_ref[b, :] + (b * _C)


def _compute_logits(x, w, bias, coords_pad):
    return pl.pallas_call(
        _logits_body,
        out_shape=(jax.ShapeDtypeStruct((_NP, _CT), jnp.int32),
                   jax.ShapeDtypeStruct((_NB, _VPAD), jnp.int32)),
    )(x, w, bias, coords_pad)


def _decode_body(tt_hbm, idx_hbm, out_hbm,
                 slice_v, idx_v, out_v0, out_v1, so0, so1):
    cid = lax.axis_index("c")
    sid = lax.axis_index("s")
    p0 = sid * _PAIR_PER_TILE

    pltpu.sync_copy(tt_hbm.at[pl.ds(p0, _PAIR_PER_TILE), :], slice_v)

    def _do_minis(out_v, width):
        @pl.loop(0, width // _MINI)
        def _mini(j):
            idxs = [idx_v[b, pl.ds(j * _MINI, _MINI)] for b in range(_NB)]
            for p in range(_PAIR_PER_TILE):
                rows = jnp.full((_MINI,), p, jnp.int32)
                acc = plsc.bitcast(
                    plsc.load_gather(slice_v, [rows, idxs[0]]), jnp.bfloat16)
                for b in range(1, _NB):
                    acc = acc + plsc.bitcast(
                        plsc.load_gather(slice_v, [rows, idxs[b]]),
                        jnp.bfloat16)
                out_v[p, pl.ds(j * _MINI, _MINI)] = plsc.bitcast(
                    acc, jnp.int32)

    bufs = (out_v0, out_v1)
    sems = (so0, so1)

    @pl.loop(0, _NFULL // 2, step=2)
    def _chunk(i):
        for b in range(2):
            ii = i + b
            k = cid + 2 * ii
            start = pl.multiple_of(k * _CHUNK, _CHUNK)
            dst = out_hbm.at[pl.ds(p0, _PAIR_PER_TILE), pl.ds(start, _CHUNK)]
            pltpu.sync_copy(idx_hbm.at[:, pl.ds(start, _CHUNK)], idx_v)

            @pl.when(ii >= 2)
            def _drain():
                pltpu.make_async_copy(bufs[b], dst, sems[b]).wait()

            _do_minis(bufs[b], _CHUNK)
            pltpu.async_copy(bufs[b], dst, sems[b])

    for b in range(2):
        dst = out_hbm.at[pl.ds(p0, _PAIR_PER_TILE), pl.ds(0, _CHUNK)]
        pltpu.make_async_copy(bufs[b], dst, sems[b]).wait()

    tstart = _NFULL * _CHUNK + cid * _TAILH
    pltpu.sync_copy(idx_hbm.at[:, pl.ds(tstart, _TAILH)],
                    idx_v.at[:, pl.ds(0, _TAILH)])
    _do_minis(out_v0, _TAILH)
    pltpu.sync_copy(out_v0.at[:, pl.ds(0, _TAILH)],
                    out_hbm.at[pl.ds(p0, _PAIR_PER_TILE),
                               pl.ds(tstart, _TAILH)])


_SC_PARAMS = pltpu.CompilerParams()
if "needs_layout_passes" in pltpu.CompilerParams.__dataclass_fields__:
    _SC_PARAMS = dataclasses.replace(_SC_PARAMS, needs_layout_passes=False)


@functools.partial(
    pl.kernel,
    out_type=jax.ShapeDtypeStruct((_NP, _VPAD), jnp.int32),
    compiler_params=_SC_PARAMS,
    mesh=plsc.VectorSubcoreMesh(core_axis_name="c", subcore_axis_name="s"),
    scratch_types=[
        pltpu.VMEM((_PAIR_PER_TILE, _CT), jnp.int32),
        pltpu.VMEM((_NB, _CHUNK), jnp.int32),
        pltpu.VMEM((_PAIR_PER_TILE, _CHUNK), jnp.int32),
        pltpu.VMEM((_PAIR_PER_TILE, _CHUNK), jnp.int32),
        pltpu.SemaphoreType.DMA,
        pltpu.SemaphoreType.DMA,
    ],
)
def _decode(tt_hbm, idx_hbm, out_hbm,
            slice_v, idx_v, out_v0, out_v1, so0, so1):
    _decode_body(tt_hbm, idx_hbm, out_hbm,
                 slice_v, idx_v, out_v0, out_v1, so0, so1)


_FIN_BLOCK = 8192


def _finish_body(in_ref, out_ref):
    xu = lax.bitcast_convert_type(in_ref[...], jnp.uint32)
    lo = lax.bitcast_convert_type(xu << 16, jnp.float32)
    hi = lax.bitcast_convert_type(xu & jnp.uint32(0xFFFF0000), jnp.float32)
    out_ref[:_NP, :] = lo
    out_ref[_NP:, :] = hi


def _finish(packed):
    grid = (_V + _FIN_BLOCK - 1) // _FIN_BLOCK
    return pl.pallas_call(
        _finish_body,
        grid=(grid,),
        in_specs=[pl.BlockSpec((_NP, _FIN_BLOCK), lambda i: (0, i))],
        out_specs=pl.BlockSpec((_N, _FIN_BLOCK), lambda i: (0, i)),
        out_shape=jax.ShapeDtypeStruct((_N, _V), jnp.float32),
    )(packed)


@jax.jit
def kernel(input, weight, bias, coordinates):
    coords_pad = jnp.concatenate(
        [coordinates,
         jnp.zeros((_NB, _VPAD - _V), jnp.int32)], axis=1)
    tt, idxp = _compute_logits(input, weight, bias, coords_pad)
    packed = _decode(tt, idxp)
    return _finish(packed)


# speedup vs baseline: 2.6601x; 1.8765x over previous
import dataclasses
import functools

import jax
import jax.numpy as jnp
from jax import lax
from jax.experimental import pallas as pl
from jax.experimental.pallas import tpu as pltpu
from jax.experimental.pallas import tpu_sc as plsc

_NB = 4
_C = 512
_F = 128
_N = 256
_NP = _N // 2
_V = 100000
_CT = _NB * _C

_NTILES = 32
_W = 64
_NCH = 49
_PER_TILE = _W * _NCH
_VPAD = _NTILES * _PER_TILE


def _logits_body(x_ref, w_ref, b_ref, c_ref, out_ref, idx_ref):
    for b in range(_NB):
        xb = x_ref[:, b * _F:(b + 1) * _F]
        wb = w_ref[b]
        acc = lax.dot_general(
            wb, xb, (((1,), (1,)), ((), ())),
            preferred_element_type=jnp.float32)
        acc = acc + b_ref[b][:, None]
        lo = lax.bitcast_convert_type(
            acc[:, :_NP].astype(jnp.bfloat16), jnp.uint16).astype(jnp.uint32)
        hi = lax.bitcast_convert_type(
            acc[:, _NP:].astype(jnp.bfloat16), jnp.uint16).astype(jnp.uint32)
        packed = lo | (hi << 16)
        out_ref[b * _C:(b + 1) * _C, :] = lax.bitcast_convert_type(
            packed, jnp.int32)
        idx_ref[b, :] = c_ref[b, :] + (b * _C)


def _compute_logits(x, w, bias, coords_pad):
    return pl.pallas_call(
        _logits_body,
        out_shape=(jax.ShapeDtypeStruct((_CT, _NP), jnp.int32),
                   jax.ShapeDtypeStruct((_NB, _VPAD), jnp.int32)),
    )(x, w, bias, coords_pad)


def _decode_body(tt_hbm, idx_hbm, out_hbm,
                 idxa, rows0, rows1, out_v0, out_v1, sg0, sg1, so0, so1):
    cid = lax.axis_index("c")
    sid = lax.axis_index("s")
    wid = sid * 2 + cid
    vb = wid * _PER_TILE

    for b in range(_NB):
        pltpu.sync_copy(idx_hbm.at[pl.ds(b * _VPAD + vb, _PER_TILE)],
                        idxa.at[pl.ds(b * _PER_TILE, _PER_TILE)])

    rows = (rows0, rows1)
    gsems = (sg0, sg1)
    osems = (so0, so1)

    def _gather_start(k, s):
        for b in range(_NB):
            pltpu.async_copy(
                tt_hbm.at[idxa.at[pl.ds(b * _PER_TILE + k * _W, _W)]],
                rows[s][b], gsems[s])

    def _gather_wait(k, s):
        for b in range(_NB):
            pltpu.make_async_copy(
                tt_hbm.at[idxa.at[pl.ds(b * _PER_TILE + k * _W, _W)]],
                rows[s][b], gsems[s]).wait()

    def _out_start(k, s):
        pltpu.async_copy(
            out_v0 if s == 0 else out_v1,
            out_hbm.at[pl.ds(vb + k * _W, _W), :], osems[s])

    def _out_wait(s):
        pltpu.make_async_copy(
            out_v0 if s == 0 else out_v1,
            out_hbm.at[pl.ds(vb, _W), :], osems[s]).wait()

    def _compute(s):
        rset = rows[s]
        out_v = out_v0 if s == 0 else out_v1

        @pl.loop(0, _W)
        def _row(v):
            for r in range(_NP // 16):
                sl = pl.ds(r * 16, 16)
                acc = plsc.bitcast(rset[0][v, sl], jnp.bfloat16)
                for b in range(1, _NB):
                    acc = acc + plsc.bitcast(rset[b][v, sl], jnp.bfloat16)
                out_v[v, sl] = plsc.bitcast(acc, jnp.int32)

    _gather_start(0, 0)

    @pl.loop(0, _NCH - 1, step=2)
    def _chunk(i):
        for s in range(2):
            k = i + s
            _gather_start(k + 1, 1 - s)
            _gather_wait(k, s)

            @pl.when(k >= 2)
            def _drain():
                _out_wait(s)

            _compute(s)
            _out_start(k, s)

    _gather_wait(_NCH - 1, 0)
    _out_wait(0)
    _compute(0)
    _out_start(_NCH - 1, 0)
    _out_wait(0)
    _out_wait(1)


_SC_PARAMS = pltpu.CompilerParams()
if "needs_layout_passes" in pltpu.CompilerParams.__dataclass_fields__:
    _SC_PARAMS = dataclasses.replace(_SC_PARAMS, needs_layout_passes=False)


@functools.partial(
    pl.kernel,
    out_type=jax.ShapeDtypeStruct((_VPAD, _NP), jnp.int32),
    compiler_params=_SC_PARAMS,
    mesh=plsc.VectorSubcoreMesh(core_axis_name="c", subcore_axis_name="s"),
    scratch_types=[
        pltpu.VMEM((_NB * _PER_TILE,), jnp.int32),
        tuple(pltpu.VMEM((_W, _NP), jnp.int32) for _ in range(_NB)),
        tuple(pltpu.VMEM((_W, _NP), jnp.int32) for _ in range(_NB)),
        pltpu.VMEM((_W, _NP), jnp.int32),
        pltpu.VMEM((_W, _NP), jnp.int32),
        pltpu.SemaphoreType.DMA,
        pltpu.SemaphoreType.DMA,
        pltpu.SemaphoreType.DMA,
        pltpu.SemaphoreType.DMA,
    ],
)
def _decode(tt_hbm, idx_hbm, out_hbm,
            idxa, rows0, rows1, out_v0, out_v1, sg0, sg1, so0, so1):
    _decode_body(tt_hbm, idx_hbm, out_hbm,
                 idxa, rows0, rows1, out_v0, out_v1, sg0, sg1, so0, so1)


_FB = 2000


def _finish_body(in_ref, out_ref):
    xu = lax.bitcast_convert_type(in_ref[...], jnp.uint32)
    lo = lax.bitcast_convert_type(xu << 16, jnp.float32)
    hi = lax.bitcast_convert_type(xu & jnp.uint32(0xFFFF0000), jnp.float32)
    out_ref[:, :_NP] = lo
    out_ref[:, _NP:] = hi


def _finish(packed):
    return pl.pallas_call(
        _finish_body,
        grid=(_V // _FB,),
        in_specs=[pl.BlockSpec((_FB, _NP), lambda i: (i, 0))],
        out_specs=pl.BlockSpec((_FB, _N), lambda i: (i, 0)),
        out_shape=jax.ShapeDtypeStruct((_V, _N), jnp.float32),
    )(packed)


@jax.jit
def kernel(input, weight, bias, coordinates):
    coords_pad = jnp.concatenate(
        [coordinates,
         jnp.zeros((_NB, _VPAD - _V), jnp.int32)], axis=1)
    tt, idxp = _compute_logits(input, weight, bias, coords_pad)
    idxf = idxp.reshape(_NB * _VPAD)
    packed = _decode(tt, idxf)
    return _finish(packed).T
